# Initial kernel scaffold; baseline (speedup 1.0000x reference)
import jax
import jax.numpy as jnp
from jax.experimental import pallas as pl


def kernel(features, nodes, nbrs, num_sample):
    raise NotImplementedError("write your pallas kernel here")


# speedup vs baseline: 2.1819x; 2.1819x over previous
import functools

import jax
import jax.numpy as jnp
from jax import lax
from jax.experimental import pallas as pl
from jax.experimental.pallas import tpu as pltpu
from jax.experimental.pallas import tpu_sc as plsc

NC = 2
NS = 16
NW = NC * NS
L = 16


def _make_sc_kernel(B, S, V, D, idx_cols):
    G_TOT = B // L
    GMAX = -(-G_TOT // NW)
    GSZ = L * S
    streams_per_group = GSZ // idx_cols
    DV = D // L

    mesh = plsc.VectorSubcoreMesh(
        core_axis_name="c", subcore_axis_name="s",
        num_cores=NC, num_subcores=NS)

    @functools.partial(
        pl.kernel,
        out_type=jax.ShapeDtypeStruct((B, D), jnp.float32),
        mesh=mesh,
        scratch_types=[
            pltpu.VMEM((GMAX * GSZ,), jnp.int32),
            pltpu.VMEM((L,), jnp.float32),
            pltpu.VMEM((GSZ, D), jnp.float32),
            pltpu.VMEM((L, D), jnp.float32),
            pltpu.SemaphoreType.DMA,
        ],
        compiler_params=pltpu.CompilerParams(needs_layout_passes=False),
    )
    def sc_kernel(feat_hbm, nbrs_hbm, valid_hbm, out_hbm,
                  nbrs_v, valid_v, rows_v, out_v, sem):
        cid = lax.axis_index("c")
        sid = lax.axis_index("s")
        wid = sid * NC + cid
        g0 = (wid * G_TOT) // NW
        g1 = ((wid + 1) * G_TOT) // NW

        pltpu.sync_copy(valid_hbm, valid_v)
        pltpu.sync_copy(nbrs_hbm.at[pl.ds(GSZ * g0, GSZ * GMAX)], nbrs_v)
        vvec = valid_v[pl.ds(0, L)]
        valid_s = [vvec[sx] for sx in range(S)]

        lanes10 = lax.iota(jnp.int32, L) * S

        def gbody(g, carry):
            gl = g - g0
            cps = [
                pltpu.async_copy(
                    feat_hbm.at[nbrs_v.at[pl.ds(gl * GSZ + j * idx_cols,
                                                idx_cols)]],
                    rows_v.at[pl.ds(j * idx_cols, idx_cols)],
                    sem)
                for j in range(streams_per_group)
            ]

            base = lanes10 + gl * GSZ
            cols = [plsc.load_gather(nbrs_v, [base + sx]) for sx in range(S)]
            w = [None] * S
            w[0] = jnp.full((L,), 1.0, jnp.float32) * valid_s[0]
            for sx in range(1, S):
                dup = cols[sx] == cols[0]
                for t in range(1, sx):
                    dup = dup | (cols[sx] == cols[t])
                w[sx] = jnp.where(dup, 0.0, valid_s[sx])
            n_unique = w[0]
            for sx in range(1, S):
                n_unique = n_unique + w[sx]
            inv = 1.0 / n_unique
            wn = [w[sx] * inv for sx in range(S)]

            for cp in cps:
                cp.wait()

            for r in range(L):
                acc = [None] * DV
                for sx in range(S):
                    wgt = wn[sx][r]
                    for d in range(DV):
                        v = wgt * rows_v[r * S + sx, pl.ds(L * d, L)]
                        acc[d] = v if sx == 0 else acc[d] + v
                for d in range(DV):
                    out_v[r, pl.ds(L * d, L)] = acc[d]

            pltpu.sync_copy(out_v, out_hbm.at[pl.ds(g * L, L)])
            return carry

        lax.fori_loop(g0, g1, gbody, 0)

    return sc_kernel


def kernel(features, nodes, nbrs, num_sample=10):
    del nodes
    B, S = nbrs.shape
    V, D = features.shape
    idx_cols = 80
    valid = (jnp.arange(L) < num_sample).astype(jnp.float32)
    nbrs_flat = nbrs.astype(jnp.int32).reshape(B * S)
    sc = _make_sc_kernel(B, S, V, D, idx_cols)
    return sc(features, nbrs_flat, valid)


# speedup vs baseline: 7.4838x; 3.4299x over previous
import functools

import jax
import jax.numpy as jnp
from jax import lax
from jax.experimental import pallas as pl
from jax.experimental.pallas import tpu as pltpu
from jax.experimental.pallas import tpu_sc as plsc

NC = 2
NS = 16
NW = NC * NS
L = 16


def _make_sc_kernel(B, S, V, D, idx_cols):
    G_TOT = B // L
    GMAX = -(-G_TOT // NW)
    GMAX += GMAX % 2
    GSZ = L * S
    NSTR = GSZ // idx_cols
    DV = D // L

    mesh = plsc.VectorSubcoreMesh(
        core_axis_name="c", subcore_axis_name="s",
        num_cores=NC, num_subcores=NS)

    @functools.partial(
        pl.kernel,
        out_type=jax.ShapeDtypeStruct((B, D), jnp.float32),
        mesh=mesh,
        scratch_types=[
            pltpu.VMEM((GMAX * GSZ,), jnp.int32),
            pltpu.VMEM((L,), jnp.float32),
            pltpu.VMEM((2 * GSZ, D), jnp.float32),
            pltpu.VMEM((2 * L, D), jnp.float32),
            pltpu.VMEM((2 * L * L,), jnp.float32),
            pltpu.SemaphoreType.DMA,
            pltpu.SemaphoreType.DMA,
            pltpu.SemaphoreType.DMA,
            pltpu.SemaphoreType.DMA,
        ],
        compiler_params=pltpu.CompilerParams(needs_layout_passes=False),
    )
    def sc_kernel(feat_hbm, nbrs_hbm, valid_hbm, out_hbm,
                  nbrs_v, valid_v, rows_v, out_v, w_v,
                  gsem0, gsem1, osem0, osem1):
        gsem = [gsem0, gsem1]
        osem = [osem0, osem1]
        cid = lax.axis_index("c")
        sid = lax.axis_index("s")
        wid = sid * NC + cid
        g0 = (wid * G_TOT) // NW
        g1m1 = ((wid + 1) * G_TOT) // NW - 1

        pltpu.sync_copy(valid_hbm, valid_v)
        pltpu.sync_copy(nbrs_hbm.at[pl.ds(GSZ * g0, GSZ * GMAX)], nbrs_v)
        vvec = valid_v[pl.ds(0, L)]
        valid_s = [vvec[sx] for sx in range(S)]

        lanes = lax.iota(jnp.int32, L)
        lanes10 = lanes * S
        lanes16 = lanes * L

        def gather_parts(gl, b):
            return [
                (feat_hbm.at[nbrs_v.at[pl.ds(gl * GSZ + j * idx_cols,
                                             idx_cols)]],
                 rows_v.at[pl.ds(b * GSZ + j * idx_cols, idx_cols)],
                 gsem[b])
                for j in range(NSTR)
            ]

        def issue_gather(i, b):
            gl = jnp.minimum(g0 + i, g1m1) - g0
            for src, dst, sem in gather_parts(gl, b):
                pltpu.async_copy(src, dst, sem)

        def process(i, k, b):
            g = jnp.minimum(g0 + i, g1m1)
            gl = g - g0

            base = lanes10 + gl * GSZ
            cols = [plsc.load_gather(nbrs_v, [base + sx]) for sx in range(S)]
            w = [None] * S
            w[0] = jnp.full((L,), 1.0, jnp.float32) * valid_s[0]
            for sx in range(1, S):
                dup = cols[sx] == cols[0]
                for t in range(1, sx):
                    dup = dup | (cols[sx] == cols[t])
                w[sx] = jnp.where(dup, 0.0, valid_s[sx])
            n_unique = w[0]
            for sx in range(1, S):
                n_unique = n_unique + w[sx]
            inv = 1.0 / n_unique
            for sx in range(S):
                plsc.store_scatter(w_v, [lanes16 + (b * L * L + sx)],
                                   w[sx] * inv)

            for src, dst, sem in gather_parts(gl, b):
                pltpu.make_async_copy(src, dst, sem).wait()

            @pl.when(k > 0)
            def _():
                pltpu.make_async_copy(
                    out_v.at[pl.ds(b * L, L)],
                    out_hbm.at[pl.ds(g * L, L)], osem[b]).wait()

            def rbody(r, c2):
                wvec = w_v[pl.ds(b * L * L + r * L, L)]
                row0 = b * GSZ + r * S
                acc = [None] * DV
                for sx in range(S):
                    wgt = wvec[sx]
                    for d in range(DV):
                        v = wgt * rows_v[row0 + sx, pl.ds(L * d, L)]
                        acc[d] = v if sx == 0 else acc[d] + v
                for d in range(DV):
                    out_v[b * L + r, pl.ds(L * d, L)] = acc[d]
                return c2

            lax.fori_loop(0, L, rbody, 0)
            pltpu.async_copy(out_v.at[pl.ds(b * L, L)],
                             out_hbm.at[pl.ds(g * L, L)], osem[b])

        issue_gather(0, 0)

        def kbody(k, carry):
            i0 = 2 * k
            issue_gather(i0 + 1, 1)
            process(i0, k, 0)

            @pl.when(k < GMAX // 2 - 1)
            def _():
                issue_gather(i0 + 2, 0)

            process(i0 + 1, k, 1)
            return carry

        lax.fori_loop(0, GMAX // 2, kbody, 0)

        for b in range(2):
            g = jnp.minimum(g0 + GMAX - 2 + b, g1m1)
            pltpu.make_async_copy(out_v.at[pl.ds(b * L, L)],
                                  out_hbm.at[pl.ds(g * L, L)], osem[b]).wait()

    return sc_kernel


def kernel(features, nodes, nbrs, num_sample=10):
    del nodes
    B, S = nbrs.shape
    V, D = features.shape
    idx_cols = 80
    valid = (jnp.arange(L) < num_sample).astype(jnp.float32)
    nbrs_flat = nbrs.astype(jnp.int32).reshape(B * S)
    sc = _make_sc_kernel(B, S, V, D, idx_cols)
    return sc(features, nbrs_flat, valid)
